# baseline (speedup 1.0000x reference)
import jax
import jax.numpy as jnp
from jax import lax
from jax.experimental import pallas as pl
from jax.experimental.pallas import tpu as pltpu
from jax.experimental.pallas import tpu_sc as plsc

BATCH = 1024
CLASSES = 100000
NUM_CORES = 2
NUM_SUBCORES = 16
NUM_WORKERS = NUM_CORES * NUM_SUBCORES
ROWS_PER_WORKER = BATCH // NUM_WORKERS
STRIPES_PER_WORKER = ROWS_PER_WORKER // 8
COL_CHUNK = 6400
FULL_CHUNKS = 15
ALIGNED_COLS = 99968
TAIL = ALIGNED_COLS - FULL_CHUNKS * COL_CHUNK
CHUNKS_PER_STRIPE = FULL_CHUNKS + 1


def _sc_copy(preds_hbm, out_hbm, buf0, buf1, si0, si1, so0, so1):
    wid = lax.axis_index("s") * NUM_CORES + lax.axis_index("c")
    row0 = wid * ROWS_PER_WORKER
    bufs = (buf0, buf1)
    in_sems = (si0, si1)
    out_sems = (so0, so1)

    def chunk_at(i):
        st, t = divmod(i, CHUNKS_PER_STRIPE)
        row = row0 + st * 8
        col = t * COL_CHUNK
        size = COL_CHUNK if t < FULL_CHUNKS else TAIL
        return row, col, size

    def buf_ref(b, size):
        return bufs[b] if size == COL_CHUNK else bufs[b].at[:, pl.ds(0, size)]

    n = STRIPES_PER_WORKER * CHUNKS_PER_STRIPE
    for i in range(n):
        b = i % 2
        if i >= 2:
            pr, pc, psz = chunk_at(i - 2)
            pltpu.make_async_copy(
                buf_ref(b, psz),
                out_hbm.at[pl.ds(pr, 8), pl.ds(pc, psz)],
                out_sems[b],
            ).wait()
        row, col, size = chunk_at(i)
        pltpu.async_copy(
            preds_hbm.at[pl.ds(row, 8), pl.ds(col, size)],
            buf_ref(b, size),
            in_sems[b],
        )
        if i >= 1:
            pr, pc, psz = chunk_at(i - 1)
            pltpu.make_async_copy(
                preds_hbm.at[pl.ds(pr, 8), pl.ds(pc, psz)],
                buf_ref(1 - b, psz),
                in_sems[1 - b],
            ).wait()
            pltpu.async_copy(
                buf_ref(1 - b, psz),
                out_hbm.at[pl.ds(pr, 8), pl.ds(pc, psz)],
                out_sems[1 - b],
            )
    pr, pc, psz = chunk_at(n - 1)
    b = (n - 1) % 2
    pltpu.make_async_copy(
        preds_hbm.at[pl.ds(pr, 8), pl.ds(pc, psz)], buf_ref(b, psz), in_sems[b]
    ).wait()
    pltpu.async_copy(
        buf_ref(b, psz), out_hbm.at[pl.ds(pr, 8), pl.ds(pc, psz)], out_sems[b]
    )
    for i in (n - 2, n - 1):
        pr, pc, psz = chunk_at(i)
        pltpu.make_async_copy(
            buf_ref(i % 2, psz),
            out_hbm.at[pl.ds(pr, 8), pl.ds(pc, psz)],
            out_sems[i % 2],
        ).wait()


TAIL_BLOCK_COLS = 128
TAIL_BLOCK_ROWS = 256


def _tail_fix(out_in_ref, preds_ref, out_ref):
    del out_in_ref
    out_ref[...] = preds_ref[...]


def kernel(preds, atoms):
    del atoms
    out = pl.kernel(
        _sc_copy,
        out_type=jax.ShapeDtypeStruct((BATCH, CLASSES), preds.dtype),
        mesh=plsc.VectorSubcoreMesh(core_axis_name="c", subcore_axis_name="s"),
        scratch_types=[
            pltpu.VMEM((8, COL_CHUNK), jnp.float32),
            pltpu.VMEM((8, COL_CHUNK), jnp.float32),
            pltpu.SemaphoreType.DMA,
            pltpu.SemaphoreType.DMA,
            pltpu.SemaphoreType.DMA,
            pltpu.SemaphoreType.DMA,
        ],
    )(preds)
    tile_col = ALIGNED_COLS // TAIL_BLOCK_COLS
    return pl.pallas_call(
        _tail_fix,
        grid=(BATCH // TAIL_BLOCK_ROWS,),
        in_specs=[
            pl.BlockSpec(memory_space=pltpu.MemorySpace.HBM),
            pl.BlockSpec((TAIL_BLOCK_ROWS, TAIL_BLOCK_COLS), lambda i: (i, tile_col)),
        ],
        out_specs=pl.BlockSpec((TAIL_BLOCK_ROWS, TAIL_BLOCK_COLS), lambda i: (i, tile_col)),
        out_shape=jax.ShapeDtypeStruct((BATCH, CLASSES), preds.dtype),
        input_output_aliases={0: 0},
    )(out, preds)


# speedup vs baseline: 1.0780x; 1.0780x over previous
import jax
import jax.numpy as jnp
from jax import lax
from jax.experimental import pallas as pl
from jax.experimental.pallas import tpu as pltpu
from jax.experimental.pallas import tpu_sc as plsc

BATCH = 1024
CLASSES = 100000
NUM_CORES = 2
NUM_SUBCORES = 16
NUM_WORKERS = NUM_CORES * NUM_SUBCORES
ROWS_PER_WORKER = BATCH // NUM_WORKERS
STRIPES_PER_WORKER = ROWS_PER_WORKER // 8
COL_CHUNK = 6400
FULL_CHUNKS = 15
ALIGNED_COLS = 99968
TAIL = ALIGNED_COLS - FULL_CHUNKS * COL_CHUNK
CHUNKS_PER_STRIPE = FULL_CHUNKS + 1


def _sc_copy(preds_hbm, out_hbm, buf0, buf1, si0, si1, so0, so1):
    wid = lax.axis_index("s") * NUM_CORES + lax.axis_index("c")
    row0 = wid * ROWS_PER_WORKER
    bufs = (buf0, buf1)
    in_sems = (si0, si1)
    out_sems = (so0, so1)

    def chunk_at(i):
        st, t = divmod(i, CHUNKS_PER_STRIPE)
        row = row0 + st * 8
        col = t * COL_CHUNK
        size = COL_CHUNK if t < FULL_CHUNKS else TAIL
        return row, col, size

    def buf_ref(b, size):
        return bufs[b] if size == COL_CHUNK else bufs[b].at[:, pl.ds(0, size)]

    n = STRIPES_PER_WORKER * CHUNKS_PER_STRIPE
    for i in range(n):
        b = i % 2
        if i >= 2:
            pr, pc, psz = chunk_at(i - 2)
            pltpu.make_async_copy(
                buf_ref(b, psz),
                out_hbm.at[pl.ds(pr, 8), pl.ds(pc, psz)],
                out_sems[b],
            ).wait()
        row, col, size = chunk_at(i)
        pltpu.async_copy(
            preds_hbm.at[pl.ds(row, 8), pl.ds(col, size)],
            buf_ref(b, size),
            in_sems[b],
        )
        if i >= 1:
            pr, pc, psz = chunk_at(i - 1)
            pltpu.make_async_copy(
                preds_hbm.at[pl.ds(pr, 8), pl.ds(pc, psz)],
                buf_ref(1 - b, psz),
                in_sems[1 - b],
            ).wait()
            pltpu.async_copy(
                buf_ref(1 - b, psz),
                out_hbm.at[pl.ds(pr, 8), pl.ds(pc, psz)],
                out_sems[1 - b],
            )
    pr, pc, psz = chunk_at(n - 1)
    b = (n - 1) % 2
    pltpu.make_async_copy(
        preds_hbm.at[pl.ds(pr, 8), pl.ds(pc, psz)], buf_ref(b, psz), in_sems[b]
    ).wait()
    pltpu.async_copy(
        buf_ref(b, psz), out_hbm.at[pl.ds(pr, 8), pl.ds(pc, psz)], out_sems[b]
    )
    for i in (n - 2, n - 1):
        pr, pc, psz = chunk_at(i)
        pltpu.make_async_copy(
            buf_ref(i % 2, psz),
            out_hbm.at[pl.ds(pr, 8), pl.ds(pc, psz)],
            out_sems[i % 2],
        ).wait()


def kernel(preds, atoms):
    del atoms
    out = pl.kernel(
        _sc_copy,
        out_type=jax.ShapeDtypeStruct((BATCH, CLASSES), preds.dtype),
        mesh=plsc.VectorSubcoreMesh(core_axis_name="c", subcore_axis_name="s"),
        scratch_types=[
            pltpu.VMEM((8, COL_CHUNK), jnp.float32),
            pltpu.VMEM((8, COL_CHUNK), jnp.float32),
            pltpu.SemaphoreType.DMA,
            pltpu.SemaphoreType.DMA,
            pltpu.SemaphoreType.DMA,
            pltpu.SemaphoreType.DMA,
        ],
    )(preds)
    return out.at[:, ALIGNED_COLS:].set(preds[:, ALIGNED_COLS:])
